# baseline (speedup 1.0000x reference)
import functools

import jax
import jax.numpy as jnp
from jax import lax
from jax.experimental import pallas as pl
from jax.experimental.pallas import tpu as pltpu
from jax.experimental.pallas import tpu_sc as plsc

_NC = 2
_NS = 16
_NW = _NC * _NS
_LANES = 16


@functools.lru_cache(maxsize=None)
def _build(n, g):
    rpw = n // _NW
    chunk = min(rpw, 2048)
    nchunks = rpw // chunk
    assert rpw * _NW == n and nchunks * chunk == rpw

    eps = jnp.float32(1e-10)
    hi = jnp.float32(1.0 - 1e-10)
    inv_step = jnp.float32(g - 1.0)
    gmax = jnp.int32(g - 1)

    mesh = plsc.VectorSubcoreMesh(core_axis_name="c", subcore_axis_name="s")

    @functools.partial(
        pl.kernel,
        mesh=mesh,
        compiler_params=pltpu.CompilerParams(
            needs_layout_passes=False, use_tc_tiling_on_sc=False),
        out_type=jax.ShapeDtypeStruct((n,), jnp.float32),
        scratch_types=[
            pltpu.VMEM((g, g), jnp.float32),
            pltpu.VMEM((chunk,), jnp.float32),
            pltpu.VMEM((chunk,), jnp.float32),
            pltpu.VMEM((chunk,), jnp.float32),
        ],
    )
    def run(obs_hbm, grid_hbm, out_hbm, grid_v, u_v, v_v, out_v):
        wid = lax.axis_index("s") * _NC + lax.axis_index("c")
        base = wid * rpw
        pltpu.sync_copy(grid_hbm, grid_v)

        def chunk_body(c, _):
            row0 = base + c * chunk
            pltpu.sync_copy(obs_hbm.at[0, pl.ds(row0, chunk)], u_v)
            pltpu.sync_copy(obs_hbm.at[1, pl.ds(row0, chunk)], v_v)

            @plsc.parallel_loop(0, chunk, step=_LANES, unroll=16)
            def vec_body(i):
                u = u_v[pl.ds(i, _LANES)]
                v = v_v[pl.ds(i, _LANES)]
                pu = jnp.minimum(jnp.maximum(u, eps), hi) * inv_step
                pv = jnp.minimum(jnp.maximum(v, eps), hi) * inv_step
                i0u = pu.astype(jnp.int32)
                i0v = pv.astype(jnp.int32)
                du = pu - i0u.astype(jnp.float32)
                dv = pv - i0v.astype(jnp.float32)
                i1u = jnp.minimum(i0u + 1, gmax)
                i1v = jnp.minimum(i0v + 1, gmax)
                g00 = plsc.load_gather(grid_v, [i0u, i0v])
                g10 = plsc.load_gather(grid_v, [i1u, i0v])
                g01 = plsc.load_gather(grid_v, [i0u, i1v])
                g11 = plsc.load_gather(grid_v, [i1u, i1v])
                res = (g00
                       + (g10 - g00) * du
                       + (g01 - g00) * dv
                       + (g11 - g01 - g10 + g00) * (du * dv))
                res = jnp.maximum(res, jnp.float32(0.0))
                out_v[pl.ds(i, _LANES)] = res

            pltpu.sync_copy(out_v, out_hbm.at[pl.ds(row0, chunk)])
            return 0

        lax.fori_loop(0, nchunks, chunk_body, 0)

    return run


def kernel(obs, pdf_grid):
    n = obs.shape[0]
    g = pdf_grid.shape[0]
    out = _build(n, g)(obs.T, pdf_grid)
    return out.reshape(n, 1)


# speedup vs baseline: 2.1262x; 2.1262x over previous
import functools

import jax
import jax.numpy as jnp
from jax import lax
from jax.experimental import pallas as pl
from jax.experimental.pallas import tpu as pltpu
from jax.experimental.pallas import tpu_sc as plsc

_NC = 2
_NS = 16
_NW = _NC * _NS
_LANES = 16


@functools.lru_cache(maxsize=None)
def _build(n, g):
    rpw = n // _NW
    chunk = min(rpw, 4096)
    nchunks = rpw // chunk
    assert rpw * _NW == n and nchunks * chunk == rpw

    eps = jnp.float32(1e-10)
    hi = jnp.float32(1.0 - 1e-10)
    inv_step = jnp.float32(g - 1.0)
    gmax = jnp.int32(g - 1)

    mesh = plsc.VectorSubcoreMesh(core_axis_name="c", subcore_axis_name="s")

    @functools.partial(
        pl.kernel,
        mesh=mesh,
        compiler_params=pltpu.CompilerParams(
            needs_layout_passes=False, use_tc_tiling_on_sc=False),
        out_type=jax.ShapeDtypeStruct((n,), jnp.float32),
        scratch_types=[
            pltpu.VMEM((g, g), jnp.float32),
            pltpu.VMEM((chunk,), jnp.float32),
            pltpu.VMEM((chunk,), jnp.float32),
            pltpu.VMEM((chunk,), jnp.float32),
        ],
    )
    def run(obs_hbm, grid_hbm, out_hbm, grid_v, u_v, v_v, out_v):
        wid = lax.axis_index("s") * _NC + lax.axis_index("c")
        base = wid * rpw
        pltpu.sync_copy(grid_hbm, grid_v)

        def chunk_body(c, _):
            row0 = base + c * chunk
            pltpu.sync_copy(obs_hbm.at[0, pl.ds(row0, chunk)], u_v)
            pltpu.sync_copy(obs_hbm.at[1, pl.ds(row0, chunk)], v_v)

            @plsc.parallel_loop(0, chunk, step=_LANES, unroll=8)
            def vec_body(i):
                u = u_v[pl.ds(i, _LANES)]
                v = v_v[pl.ds(i, _LANES)]
                pu = jnp.minimum(jnp.maximum(u, eps), hi) * inv_step
                pv = jnp.minimum(jnp.maximum(v, eps), hi) * inv_step
                i0u = pu.astype(jnp.int32)
                i0v = pv.astype(jnp.int32)
                du = pu - i0u.astype(jnp.float32)
                dv = pv - i0v.astype(jnp.float32)
                i1u = jnp.minimum(i0u + 1, gmax)
                i1v = jnp.minimum(i0v + 1, gmax)
                g00 = plsc.load_gather(grid_v, [i0u, i0v])
                g10 = plsc.load_gather(grid_v, [i1u, i0v])
                g01 = plsc.load_gather(grid_v, [i0u, i1v])
                g11 = plsc.load_gather(grid_v, [i1u, i1v])
                res = (g00
                       + (g10 - g00) * du
                       + (g01 - g00) * dv
                       + (g11 - g01 - g10 + g00) * (du * dv))
                res = jnp.maximum(res, jnp.float32(0.0))
                out_v[pl.ds(i, _LANES)] = res

            pltpu.sync_copy(out_v, out_hbm.at[pl.ds(row0, chunk)])
            return 0

        lax.fori_loop(0, nchunks, chunk_body, 0)

    return run


def kernel(obs, pdf_grid):
    n = obs.shape[0]
    g = pdf_grid.shape[0]
    out = _build(n, g)(obs.T, pdf_grid)
    return out.reshape(n, 1)


# speedup vs baseline: 2.3117x; 1.0872x over previous
import functools

import jax
import jax.numpy as jnp
from jax import lax
from jax.experimental import pallas as pl
from jax.experimental.pallas import tpu as pltpu
from jax.experimental.pallas import tpu_sc as plsc

_NC = 2
_NS = 16
_NW = _NC * _NS
_LANES = 16


@functools.lru_cache(maxsize=None)
def _build(n, g):
    rpw = n // _NW
    chunk = min(rpw, 8192)
    nchunks = rpw // chunk
    assert rpw * _NW == n and nchunks * chunk == rpw

    eps = jnp.float32(1e-10)
    hi = jnp.float32(1.0 - 1e-10)
    inv_step = jnp.float32(g - 1.0)
    gmax = jnp.int32(g - 1)

    mesh = plsc.VectorSubcoreMesh(core_axis_name="c", subcore_axis_name="s")

    @functools.partial(
        pl.kernel,
        mesh=mesh,
        compiler_params=pltpu.CompilerParams(
            needs_layout_passes=False, use_tc_tiling_on_sc=False),
        out_type=jax.ShapeDtypeStruct((n,), jnp.float32),
        scratch_types=[
            pltpu.VMEM((g, g), jnp.float32),
            pltpu.VMEM((chunk,), jnp.float32),
            pltpu.VMEM((chunk,), jnp.float32),
            pltpu.VMEM((chunk,), jnp.float32),
        ],
    )
    def run(obs_hbm, grid_hbm, out_hbm, grid_v, u_v, v_v, out_v):
        wid = lax.axis_index("s") * _NC + lax.axis_index("c")
        base = wid * rpw
        pltpu.sync_copy(grid_hbm, grid_v)

        def chunk_body(c, _):
            row0 = base + c * chunk
            pltpu.sync_copy(obs_hbm.at[0, pl.ds(row0, chunk)], u_v)
            pltpu.sync_copy(obs_hbm.at[1, pl.ds(row0, chunk)], v_v)

            @plsc.parallel_loop(0, chunk, step=_LANES, unroll=8)
            def vec_body(i):
                u = u_v[pl.ds(i, _LANES)]
                v = v_v[pl.ds(i, _LANES)]
                pu = jnp.minimum(jnp.maximum(u, eps), hi) * inv_step
                pv = jnp.minimum(jnp.maximum(v, eps), hi) * inv_step
                i0u = pu.astype(jnp.int32)
                i0v = pv.astype(jnp.int32)
                du = pu - i0u.astype(jnp.float32)
                dv = pv - i0v.astype(jnp.float32)
                i1u = jnp.minimum(i0u + 1, gmax)
                i1v = jnp.minimum(i0v + 1, gmax)
                g00 = plsc.load_gather(grid_v, [i0u, i0v])
                g10 = plsc.load_gather(grid_v, [i1u, i0v])
                g01 = plsc.load_gather(grid_v, [i0u, i1v])
                g11 = plsc.load_gather(grid_v, [i1u, i1v])
                res = (g00
                       + (g10 - g00) * du
                       + (g01 - g00) * dv
                       + (g11 - g01 - g10 + g00) * (du * dv))
                res = jnp.maximum(res, jnp.float32(0.0))
                out_v[pl.ds(i, _LANES)] = res

            pltpu.sync_copy(out_v, out_hbm.at[pl.ds(row0, chunk)])
            return 0

        lax.fori_loop(0, nchunks, chunk_body, 0)

    return run


def kernel(obs, pdf_grid):
    n = obs.shape[0]
    g = pdf_grid.shape[0]
    out = _build(n, g)(obs.T, pdf_grid)
    return out.reshape(n, 1)


# speedup vs baseline: 2.4303x; 1.0513x over previous
import functools

import jax
import jax.numpy as jnp
from jax import lax
from jax.experimental import pallas as pl
from jax.experimental.pallas import tpu as pltpu
from jax.experimental.pallas import tpu_sc as plsc

_NC = 2
_NS = 16
_NW = _NC * _NS
_LANES = 16


@functools.lru_cache(maxsize=None)
def _build(n, g):
    rpw = n // _NW
    chunk = min(rpw, 16384)
    nchunks = rpw // chunk
    assert rpw * _NW == n and nchunks * chunk == rpw

    eps = jnp.float32(1e-10)
    hi = jnp.float32(1.0 - 1e-10)
    inv_step = jnp.float32(g - 1.0)
    gmax = jnp.int32(g - 1)

    mesh = plsc.VectorSubcoreMesh(core_axis_name="c", subcore_axis_name="s")

    @functools.partial(
        pl.kernel,
        mesh=mesh,
        compiler_params=pltpu.CompilerParams(
            needs_layout_passes=False, use_tc_tiling_on_sc=False),
        out_type=jax.ShapeDtypeStruct((n,), jnp.float32),
        scratch_types=[
            pltpu.VMEM((g, g), jnp.float32),
            pltpu.VMEM((chunk,), jnp.float32),
            pltpu.VMEM((chunk,), jnp.float32),
            pltpu.VMEM((chunk,), jnp.float32),
        ],
    )
    def run(obs_hbm, grid_hbm, out_hbm, grid_v, u_v, v_v, out_v):
        wid = lax.axis_index("s") * _NC + lax.axis_index("c")
        base = wid * rpw
        pltpu.sync_copy(grid_hbm, grid_v)

        def chunk_body(c, _):
            row0 = base + c * chunk
            pltpu.sync_copy(obs_hbm.at[0, pl.ds(row0, chunk)], u_v)
            pltpu.sync_copy(obs_hbm.at[1, pl.ds(row0, chunk)], v_v)

            @plsc.parallel_loop(0, chunk, step=_LANES, unroll=8)
            def vec_body(i):
                u = u_v[pl.ds(i, _LANES)]
                v = v_v[pl.ds(i, _LANES)]
                pu = jnp.minimum(jnp.maximum(u, eps), hi) * inv_step
                pv = jnp.minimum(jnp.maximum(v, eps), hi) * inv_step
                i0u = pu.astype(jnp.int32)
                i0v = pv.astype(jnp.int32)
                du = pu - i0u.astype(jnp.float32)
                dv = pv - i0v.astype(jnp.float32)
                i1u = jnp.minimum(i0u + 1, gmax)
                i1v = jnp.minimum(i0v + 1, gmax)
                g00 = plsc.load_gather(grid_v, [i0u, i0v])
                g10 = plsc.load_gather(grid_v, [i1u, i0v])
                g01 = plsc.load_gather(grid_v, [i0u, i1v])
                g11 = plsc.load_gather(grid_v, [i1u, i1v])
                res = (g00
                       + (g10 - g00) * du
                       + (g01 - g00) * dv
                       + (g11 - g01 - g10 + g00) * (du * dv))
                res = jnp.maximum(res, jnp.float32(0.0))
                out_v[pl.ds(i, _LANES)] = res

            pltpu.sync_copy(out_v, out_hbm.at[pl.ds(row0, chunk)])
            return 0

        lax.fori_loop(0, nchunks, chunk_body, 0)

    return run


def kernel(obs, pdf_grid):
    n = obs.shape[0]
    g = pdf_grid.shape[0]
    out = _build(n, g)(obs.T, pdf_grid)
    return out.reshape(n, 1)


# speedup vs baseline: 3.2553x; 1.3394x over previous
import functools

import jax
import jax.numpy as jnp
from jax import lax
from jax.experimental import pallas as pl
from jax.experimental.pallas import tpu as pltpu
from jax.experimental.pallas import tpu_sc as plsc

_NC = 2
_NS = 16
_NW = _NC * _NS
_LANES = 16


@functools.lru_cache(maxsize=None)
def _build(n, g):
    rpw = n // _NW
    chunk = min(rpw, 8192)
    nchunks = rpw // chunk
    assert rpw * _NW == n and nchunks * chunk == rpw

    eps = jnp.float32(1e-10)
    hi = jnp.float32(1.0 - 1e-10)
    inv_step = jnp.float32(g - 1.0)
    gmax = jnp.int32(g - 1)

    mesh = plsc.VectorSubcoreMesh(core_axis_name="c", subcore_axis_name="s")

    @functools.partial(
        pl.kernel,
        mesh=mesh,
        compiler_params=pltpu.CompilerParams(
            needs_layout_passes=False, use_tc_tiling_on_sc=False),
        out_type=jax.ShapeDtypeStruct((n,), jnp.float32),
        scratch_types=[
            pltpu.VMEM((g, g), jnp.float32),
            pltpu.VMEM((chunk,), jnp.float32),
            pltpu.VMEM((chunk,), jnp.float32),
            pltpu.VMEM((chunk,), jnp.float32),
            pltpu.VMEM((chunk,), jnp.float32),
            pltpu.VMEM((chunk,), jnp.float32),
            pltpu.VMEM((chunk,), jnp.float32),
            pltpu.SemaphoreType.DMA,
            pltpu.SemaphoreType.DMA,
            pltpu.SemaphoreType.DMA,
            pltpu.SemaphoreType.DMA,
        ],
    )
    def run(obs_hbm, grid_hbm, out_hbm, grid_v,
            u0, v0, o0, u1, v1, o1, si0, si1, so0, so1):
        wid = lax.axis_index("s") * _NC + lax.axis_index("c")
        base = wid * rpw
        pltpu.sync_copy(grid_hbm, grid_v)

        bufs = ((u0, v0, o0, si0, so0), (u1, v1, o1, si1, so1))

        def start_in(c):
            u_v, v_v, _, si, _ = bufs[c % 2]
            row0 = base + c * chunk
            pltpu.async_copy(obs_hbm.at[0, pl.ds(row0, chunk)], u_v, si)
            pltpu.async_copy(obs_hbm.at[1, pl.ds(row0, chunk)], v_v, si)

        def wait_in(c):
            u_v, v_v, _, si, _ = bufs[c % 2]
            row0 = base + c * chunk
            pltpu.make_async_copy(obs_hbm.at[0, pl.ds(row0, chunk)], u_v, si).wait()
            pltpu.make_async_copy(obs_hbm.at[1, pl.ds(row0, chunk)], v_v, si).wait()

        def start_out(c):
            _, _, o_v, _, so = bufs[c % 2]
            row0 = base + c * chunk
            pltpu.async_copy(o_v, out_hbm.at[pl.ds(row0, chunk)], so)

        def wait_out(c):
            _, _, o_v, _, so = bufs[c % 2]
            row0 = base + c * chunk
            pltpu.make_async_copy(o_v, out_hbm.at[pl.ds(row0, chunk)], so).wait()

        def compute(c):
            u_v, v_v, o_v, _, _ = bufs[c % 2]

            @plsc.parallel_loop(0, chunk, step=_LANES, unroll=4)
            def vec_body(i):
                u = u_v[pl.ds(i, _LANES)]
                v = v_v[pl.ds(i, _LANES)]
                pu = jnp.minimum(jnp.maximum(u, eps), hi) * inv_step
                pv = jnp.minimum(jnp.maximum(v, eps), hi) * inv_step
                i0u = pu.astype(jnp.int32)
                i0v = pv.astype(jnp.int32)
                du = pu - i0u.astype(jnp.float32)
                dv = pv - i0v.astype(jnp.float32)
                i1u = jnp.minimum(i0u + 1, gmax)
                i1v = jnp.minimum(i0v + 1, gmax)
                g00 = plsc.load_gather(grid_v, [i0u, i0v])
                g10 = plsc.load_gather(grid_v, [i1u, i0v])
                g01 = plsc.load_gather(grid_v, [i0u, i1v])
                g11 = plsc.load_gather(grid_v, [i1u, i1v])
                res = (g00
                       + (g10 - g00) * du
                       + (g01 - g00) * dv
                       + (g11 - g01 - g10 + g00) * (du * dv))
                res = jnp.maximum(res, jnp.float32(0.0))
                o_v[pl.ds(i, _LANES)] = res

        start_in(0)
        for c in range(nchunks):
            if c + 1 < nchunks:
                start_in(c + 1)
            wait_in(c)
            if c >= 2:
                wait_out(c - 2)
            compute(c)
            start_out(c)
        wait_out(nchunks - 2)
        wait_out(nchunks - 1)

    return run


def kernel(obs, pdf_grid):
    n = obs.shape[0]
    g = pdf_grid.shape[0]
    out = _build(n, g)(obs.T, pdf_grid)
    return out.reshape(n, 1)


# speedup vs baseline: 3.7699x; 1.1581x over previous
import functools

import jax
import jax.numpy as jnp
from jax import lax
from jax.experimental import pallas as pl
from jax.experimental.pallas import tpu as pltpu
from jax.experimental.pallas import tpu_sc as plsc

_NC = 2
_NS = 16
_NW = _NC * _NS
_LANES = 16


@functools.lru_cache(maxsize=None)
def _build(n, g):
    rpw = n // _NW
    chunk = min(rpw, 8192)
    nchunks = rpw // chunk
    assert rpw * _NW == n and nchunks * chunk == rpw

    eps = jnp.float32(1e-10)
    hi = jnp.float32(1.0 - 1e-10)
    inv_step = jnp.float32(g - 1.0)
    gp = g + 1

    mesh = plsc.VectorSubcoreMesh(core_axis_name="c", subcore_axis_name="s")

    @functools.partial(
        pl.kernel,
        mesh=mesh,
        compiler_params=pltpu.CompilerParams(
            needs_layout_passes=False, use_tc_tiling_on_sc=False),
        out_type=jax.ShapeDtypeStruct((n,), jnp.float32),
        scratch_types=[
            pltpu.VMEM((gp * gp,), jnp.float32),
            pltpu.VMEM((chunk,), jnp.float32),
            pltpu.VMEM((chunk,), jnp.float32),
            pltpu.VMEM((chunk,), jnp.float32),
            pltpu.VMEM((chunk,), jnp.float32),
            pltpu.VMEM((chunk,), jnp.float32),
            pltpu.VMEM((chunk,), jnp.float32),
            pltpu.SemaphoreType.DMA,
            pltpu.SemaphoreType.DMA,
            pltpu.SemaphoreType.DMA,
            pltpu.SemaphoreType.DMA,
        ],
    )
    def run(obs_hbm, grid_hbm, out_hbm, grid_v,
            u0, v0, o0, u1, v1, o1, si0, si1, so0, so1):
        wid = lax.axis_index("s") * _NC + lax.axis_index("c")
        base = wid * rpw
        pltpu.sync_copy(grid_hbm, grid_v)

        bufs = ((u0, v0, o0, si0, so0), (u1, v1, o1, si1, so1))

        def start_in(c):
            u_v, v_v, _, si, _ = bufs[c % 2]
            row0 = base + c * chunk
            pltpu.async_copy(obs_hbm.at[0, pl.ds(row0, chunk)], u_v, si)
            pltpu.async_copy(obs_hbm.at[1, pl.ds(row0, chunk)], v_v, si)

        def wait_in(c):
            u_v, v_v, _, si, _ = bufs[c % 2]
            row0 = base + c * chunk
            pltpu.make_async_copy(obs_hbm.at[0, pl.ds(row0, chunk)], u_v, si).wait()
            pltpu.make_async_copy(obs_hbm.at[1, pl.ds(row0, chunk)], v_v, si).wait()

        def start_out(c):
            _, _, o_v, _, so = bufs[c % 2]
            row0 = base + c * chunk
            pltpu.async_copy(o_v, out_hbm.at[pl.ds(row0, chunk)], so)

        def wait_out(c):
            _, _, o_v, _, so = bufs[c % 2]
            row0 = base + c * chunk
            pltpu.make_async_copy(o_v, out_hbm.at[pl.ds(row0, chunk)], so).wait()

        def compute(c):
            u_v, v_v, o_v, _, _ = bufs[c % 2]

            @plsc.parallel_loop(0, chunk, step=_LANES, unroll=8)
            def vec_body(i):
                u = u_v[pl.ds(i, _LANES)]
                v = v_v[pl.ds(i, _LANES)]
                pu = jnp.minimum(jnp.maximum(u, eps), hi) * inv_step
                pv = jnp.minimum(jnp.maximum(v, eps), hi) * inv_step
                i0u = pu.astype(jnp.int32)
                i0v = pv.astype(jnp.int32)
                du = pu - i0u.astype(jnp.float32)
                dv = pv - i0v.astype(jnp.float32)
                fi = i0u * jnp.int32(gp) + i0v
                g00 = plsc.load_gather(grid_v, [fi])
                g01 = plsc.load_gather(grid_v, [fi + jnp.int32(1)])
                g10 = plsc.load_gather(grid_v, [fi + jnp.int32(gp)])
                g11 = plsc.load_gather(grid_v, [fi + jnp.int32(gp + 1)])
                a = g00 + (g01 - g00) * dv
                b = g10 + (g11 - g10) * dv
                res = a + (b - a) * du
                res = jnp.maximum(res, jnp.float32(0.0))
                o_v[pl.ds(i, _LANES)] = res

        start_in(0)
        for c in range(nchunks):
            if c + 1 < nchunks:
                start_in(c + 1)
            wait_in(c)
            if c >= 2:
                wait_out(c - 2)
            compute(c)
            start_out(c)
        wait_out(nchunks - 2)
        wait_out(nchunks - 1)

    return run


def kernel(obs, pdf_grid):
    n = obs.shape[0]
    g = pdf_grid.shape[0]
    gpad = jnp.pad(pdf_grid, ((0, 1), (0, 1)), mode="edge").reshape(-1)
    out = _build(n, g)(obs.T, gpad)
    return out.reshape(n, 1)
